# baseline (speedup 1.0000x reference)
import functools

import jax
import jax.numpy as jnp
from jax import lax
from jax.experimental import pallas as pl
from jax.experimental.pallas import tpu as pltpu
from jax.experimental.pallas import tpu_sc as plsc

NC = 2
NS = 16
NW = NC * NS
L = 16
CHUNK = 1024


@functools.lru_cache(maxsize=None)
def _make(nchunk, vocab, dim):
    mesh = plsc.VectorSubcoreMesh(core_axis_name="c", subcore_axis_name="s")
    assert nchunk % 2 == 0

    @functools.partial(
        pl.kernel,
        mesh=mesh,
        out_type=jax.ShapeDtypeStruct((NW, nchunk, CHUNK, dim), jnp.float32),
        compiler_params=pltpu.CompilerParams(
            needs_layout_passes=False, use_tc_tiling_on_sc=False),
        scratch_types=[
            pltpu.VMEM((2, CHUNK), jnp.int32),
            pltpu.VMEM((2, CHUNK, dim), jnp.float32),
            pltpu.VMEM((vocab, dim), jnp.float32),
            pltpu.SemaphoreType.DMA,
            pltpu.SemaphoreType.DMA,
            pltpu.SemaphoreType.DMA,
            pltpu.SemaphoreType.DMA,
        ],
    )
    def k(idx_hbm, table_hbm, out_hbm, idx_v, rows_v, table_v,
          isem0, isem1, osem0, osem1):
        wid = lax.axis_index("s") * NC + lax.axis_index("c")
        isems = (isem0, isem1)
        osems = (osem0, osem1)

        pltpu.sync_copy(table_hbm, table_v)

        iota = lax.iota(jnp.int32, L)

        def load_idx(c, b):
            pltpu.async_copy(idx_hbm.at[wid, c], idx_v.at[b], isems[b])

        def wait_idx(b):
            pltpu.make_async_copy(idx_hbm.at[wid, 0], idx_v.at[b],
                                  isems[b]).wait()

        def compute(b):
            rows = rows_v.at[b]

            @plsc.parallel_loop(0, CHUNK // L, unroll=4)
            def g_body(g):
                idxs = idx_v[b, pl.ds(g * L, L)]
                dst = g * L + iota
                for d in range(dim):
                    dcol = jnp.full((L,), d, jnp.int32)
                    col = plsc.load_gather(table_v, [idxs, dcol])
                    plsc.store_scatter(rows, [dst, dcol], col)

        def start_out(c, b):
            pltpu.async_copy(rows_v.at[b], out_hbm.at[wid, c], osems[b])

        def wait_out(b):
            pltpu.make_async_copy(rows_v.at[b], out_hbm.at[wid, 0],
                                  osems[b]).wait()

        load_idx(0, 0)
        ng = nchunk // 2

        def body(g, carry):
            c = g * 2

            wait_idx(0)
            load_idx(c + 1, 1)

            @pl.when(g > 0)
            def _():
                wait_out(0)

            compute(0)
            start_out(c, 0)

            wait_idx(1)

            @pl.when(g < ng - 1)
            def _():
                load_idx(c + 2, 0)

            @pl.when(g > 0)
            def _():
                wait_out(1)

            compute(1)
            start_out(c + 1, 1)
            return carry

        lax.fori_loop(0, ng, body, 0)
        wait_out(0)
        wait_out(1)

    return k


def kernel(indices, table):
    n, m = indices.shape
    vocab, dim = table.shape
    b = n * m
    nchunk = b // (NW * CHUNK)
    idx = indices.astype(jnp.int32).reshape(NW, nchunk, CHUNK)
    out = _make(nchunk, vocab, dim)(idx, table)
    return out.reshape(n, m, dim)


# speedup vs baseline: 1.1863x; 1.1863x over previous
import functools

import jax
import jax.numpy as jnp
from jax import lax
from jax.experimental import pallas as pl
from jax.experimental.pallas import tpu as pltpu
from jax.experimental.pallas import tpu_sc as plsc

NC = 2
NS = 16
NW = NC * NS
L = 16
CHUNK = 1024


@functools.lru_cache(maxsize=None)
def _make(nchunk, vocab, dim):
    mesh = plsc.VectorSubcoreMesh(core_axis_name="c", subcore_axis_name="s")
    assert nchunk % 2 == 0

    @functools.partial(
        pl.kernel,
        mesh=mesh,
        out_type=jax.ShapeDtypeStruct((NW, nchunk, CHUNK, dim), jnp.float32),
        compiler_params=pltpu.CompilerParams(
            needs_layout_passes=False, use_tc_tiling_on_sc=False),
        scratch_types=[
            pltpu.VMEM((2, CHUNK), jnp.int32),
            pltpu.VMEM((2, CHUNK, dim), jnp.float32),
            pltpu.VMEM((vocab, dim), jnp.float32),
            pltpu.SemaphoreType.DMA,
            pltpu.SemaphoreType.DMA,
            pltpu.SemaphoreType.DMA,
            pltpu.SemaphoreType.DMA,
        ],
    )
    def k(idx_hbm, table_hbm, out_hbm, idx_v, rows_v, table_v,
          isem0, isem1, osem0, osem1):
        wid = lax.axis_index("s") * NC + lax.axis_index("c")
        isems = (isem0, isem1)
        osems = (osem0, osem1)

        pltpu.sync_copy(table_hbm, table_v)

        iota = lax.iota(jnp.int32, L)

        def load_idx(c, b):
            pltpu.async_copy(idx_hbm.at[wid, c], idx_v.at[b], isems[b])

        def wait_idx(b):
            pltpu.make_async_copy(idx_hbm.at[wid, 0], idx_v.at[b],
                                  isems[b]).wait()

        def compute(b):
            rows = rows_v.at[b]

            @plsc.parallel_loop(0, CHUNK // L, unroll=2)
            def g_body(g):
                idxs = idx_v[b, pl.ds(g * L, L)]
                dst = g * L + iota

                @plsc.parallel_loop(0, dim, unroll=8)
                def d_body(d):
                    dcol = jnp.full((L,), d, jnp.int32)
                    col = plsc.load_gather(table_v, [idxs, dcol])
                    plsc.store_scatter(rows, [dst, dcol], col)

        def start_out(c, b):
            pltpu.async_copy(rows_v.at[b], out_hbm.at[wid, c], osems[b])

        def wait_out(b):
            pltpu.make_async_copy(rows_v.at[b], out_hbm.at[wid, 0],
                                  osems[b]).wait()

        load_idx(0, 0)
        ng = nchunk // 2

        def body(g, carry):
            c = g * 2

            wait_idx(0)
            load_idx(c + 1, 1)

            @pl.when(g > 0)
            def _():
                wait_out(0)

            compute(0)
            start_out(c, 0)

            wait_idx(1)

            @pl.when(g < ng - 1)
            def _():
                load_idx(c + 2, 0)

            @pl.when(g > 0)
            def _():
                wait_out(1)

            compute(1)
            start_out(c + 1, 1)
            return carry

        lax.fori_loop(0, ng, body, 0)
        wait_out(0)
        wait_out(1)

    return k


def kernel(indices, table):
    n, m = indices.shape
    vocab, dim = table.shape
    b = n * m
    nchunk = b // (NW * CHUNK)
    idx = indices.astype(jnp.int32).reshape(NW, nchunk, CHUNK)
    out = _make(nchunk, vocab, dim)(idx, table)
    return out.reshape(n, m, dim)


# speedup vs baseline: 11.2418x; 9.4763x over previous
import functools

import jax
import jax.numpy as jnp
from jax import lax
from jax.experimental import pallas as pl
from jax.experimental.pallas import tpu as pltpu
from jax.experimental.pallas import tpu_sc as plsc

NC = 2
NS = 16
NW = NC * NS
L = 16
JB = 40


@functools.lru_cache(maxsize=None)
def _make(nj, ni, vocab, dim):
    mesh = plsc.VectorSubcoreMesh(core_axis_name="c", subcore_axis_name="s")
    iw = ni // NW
    gi_n = iw // L
    njb = nj // JB

    @functools.partial(
        pl.kernel,
        mesh=mesh,
        out_type=jax.ShapeDtypeStruct((nj, dim, ni), jnp.float32),
        compiler_params=pltpu.CompilerParams(
            needs_layout_passes=False, use_tc_tiling_on_sc=True),
        scratch_types=[
            pltpu.VMEM((2, JB, iw), jnp.int32),
            pltpu.VMEM((2, dim, iw), jnp.float32),
            pltpu.VMEM((dim, vocab), jnp.float32),
            pltpu.SemaphoreType.DMA,
            pltpu.SemaphoreType.DMA,
            pltpu.SemaphoreType.DMA,
            pltpu.SemaphoreType.DMA,
        ],
    )
    def k(idx_hbm, table_hbm, out_hbm, idx_v, rows_v, table_v,
          isem0, isem1, osem0, osem1):
        wid = lax.axis_index("s") * NC + lax.axis_index("c")
        ibase = wid * iw
        isems = (isem0, isem1)
        osems = (osem0, osem1)

        pltpu.sync_copy(table_hbm, table_v)

        def load_idx(jb, b):
            pltpu.async_copy(
                idx_hbm.at[pl.ds(jb * JB, JB), pl.ds(ibase, iw)],
                idx_v.at[b], isems[b])

        def wait_idx(b):
            pltpu.make_async_copy(
                idx_hbm.at[pl.ds(0, JB), pl.ds(ibase, iw)],
                idx_v.at[b], isems[b]).wait()

        def compute(ib, jj, rb):
            rows = rows_v.at[rb]

            @plsc.parallel_loop(0, gi_n, unroll=2)
            def gi_body(gi):
                idxs = idx_v[ib, jj, pl.ds(gi * L, L)]

                @plsc.parallel_loop(0, dim, unroll=8)
                def d_body(d):
                    dfull = jnp.full((L,), d, jnp.int32)
                    val = plsc.load_gather(table_v, [dfull, idxs])
                    rows[d, pl.ds(gi * L, L)] = val

        def start_out(j, rb):
            pltpu.async_copy(
                rows_v.at[rb],
                out_hbm.at[j, pl.ds(0, dim), pl.ds(ibase, iw)],
                osems[rb])

        def wait_out(rb):
            pltpu.make_async_copy(
                rows_v.at[rb],
                out_hbm.at[0, pl.ds(0, dim), pl.ds(ibase, iw)],
                osems[rb]).wait()

        load_idx(0, 0)
        wait_idx(0)
        for jb in range(njb):
            ib = jb % 2
            if jb + 1 < njb:
                load_idx(jb + 1, 1 - ib)

            def body(t, carry, jb=jb, ib=ib):
                j = jb * JB + 2 * t
                if jb == 0:
                    @pl.when(t > 0)
                    def _():
                        wait_out(0)
                else:
                    wait_out(0)
                compute(ib, 2 * t, 0)
                start_out(j, 0)
                if jb == 0:
                    @pl.when(t > 0)
                    def _():
                        wait_out(1)
                else:
                    wait_out(1)
                compute(ib, 2 * t + 1, 1)
                start_out(j + 1, 1)
                return carry

            lax.fori_loop(0, JB // 2, body, 0)
            if jb + 1 < njb:
                wait_idx(1 - ib)
        wait_out(0)
        wait_out(1)

    return k


def kernel(indices, table):
    n, m = indices.shape
    vocab, dim = table.shape
    idx_t = indices.astype(jnp.int32).T
    table_t = table.T
    out_t = _make(m, n, vocab, dim)(idx_t, table_t)
    return out_t.transpose(2, 0, 1)
